# Initial kernel scaffold; baseline (speedup 1.0000x reference)
import jax
import jax.numpy as jnp
from jax.experimental import pallas as pl


def kernel(idx, W_train, W_frozen):
    raise NotImplementedError("write your pallas kernel here")


# speedup vs baseline: 8.1573x; 8.1573x over previous
import functools

import jax
import jax.numpy as jnp
from jax import lax
from jax.experimental import pallas as pl
from jax.experimental.pallas import tpu as pltpu
from jax.experimental.pallas import tpu_sc as plsc

NC, NS = 2, 16
NW = NC * NS
D = 32
IDX_BLK = 128
BLKS_PER_CHUNK = 8
CHUNK = IDX_BLK * BLKS_PER_CHUNK


def _sc_gather(table, idx2d):
    n = idx2d.shape[0] * IDX_BLK
    per_w = n // NW
    n_chunks = per_w // CHUNK
    blk_rows_per_chunk = BLKS_PER_CHUNK

    mesh = plsc.VectorSubcoreMesh(
        core_axis_name="c", subcore_axis_name="s",
        num_cores=NC, num_subcores=NS)

    @functools.partial(
        pl.kernel,
        out_type=jax.ShapeDtypeStruct((n, D), jnp.float32),
        mesh=mesh,
        scratch_types=[
            pltpu.VMEM((BLKS_PER_CHUNK, IDX_BLK), jnp.int32),
            pltpu.VMEM((CHUNK, D), jnp.float32),
            pltpu.SemaphoreType.DMA,
        ],
        compiler_params=pltpu.CompilerParams(use_tc_tiling_on_sc=False),
    )
    def k(table_hbm, idx_hbm, out_hbm, idx_v, rows_v, sem):
        wid = lax.axis_index("s") * NC + lax.axis_index("c")
        chunk0 = wid * n_chunks

        def step(i, carry):
            c = chunk0 + i
            pltpu.sync_copy(
                idx_hbm.at[pl.ds(c * blk_rows_per_chunk, blk_rows_per_chunk), :],
                idx_v)
            copies = [
                pltpu.async_copy(
                    table_hbm.at[idx_v.at[b]],
                    rows_v.at[pl.ds(b * IDX_BLK, IDX_BLK), :],
                    sem)
                for b in range(BLKS_PER_CHUNK)
            ]
            for cp in copies:
                cp.wait()
            pltpu.sync_copy(rows_v, out_hbm.at[pl.ds(c * CHUNK, CHUNK), :])
            return carry

        lax.fori_loop(0, n_chunks, step, 0)

    return k(table, idx2d)


def kernel(idx, W_train, W_frozen):
    table = jnp.concatenate([W_train, W_frozen], axis=0)
    flat = idx.reshape(-1)
    idx2d = flat.reshape(-1, IDX_BLK)
    out = _sc_gather(table, idx2d)
    return out.reshape(idx.shape + (D,))


# speedup vs baseline: 8.5256x; 1.0451x over previous
import functools

import jax
import jax.numpy as jnp
from jax import lax
from jax.experimental import pallas as pl
from jax.experimental.pallas import tpu as pltpu
from jax.experimental.pallas import tpu_sc as plsc

NC, NS = 2, 16
NW = NC * NS
D = 32
IDX_BLK = 128
BLKS_PER_CHUNK = 8
CHUNK = IDX_BLK * BLKS_PER_CHUNK
NBUF = 2


def _sc_gather(table, idx2d):
    n = idx2d.shape[0] * IDX_BLK
    per_w = n // NW
    n_chunks = per_w // CHUNK

    mesh = plsc.VectorSubcoreMesh(
        core_axis_name="c", subcore_axis_name="s",
        num_cores=NC, num_subcores=NS)

    @functools.partial(
        pl.kernel,
        out_type=jax.ShapeDtypeStruct((n, D), jnp.float32),
        mesh=mesh,
        scratch_types=[
            pltpu.VMEM((NBUF, BLKS_PER_CHUNK, IDX_BLK), jnp.int32),
            pltpu.VMEM((NBUF, CHUNK, D), jnp.float32),
            pltpu.SemaphoreType.DMA((NBUF,)),
            pltpu.SemaphoreType.DMA((NBUF,)),
            pltpu.SemaphoreType.DMA,
        ],
        compiler_params=pltpu.CompilerParams(use_tc_tiling_on_sc=False),
    )
    def k(table_hbm, idx_hbm, out_hbm, idx_v, rows_v, sem_idx, sem_out, sem_g):
        wid = lax.axis_index("s") * NC + lax.axis_index("c")
        chunk0 = wid * n_chunks

        def idx_copy(c, b):
            return pltpu.make_async_copy(
                idx_hbm.at[pl.ds((chunk0 + c) * BLKS_PER_CHUNK, BLKS_PER_CHUNK), :],
                idx_v.at[b], sem_idx.at[b])

        def out_copy(c, b):
            return pltpu.make_async_copy(
                rows_v.at[b], out_hbm.at[pl.ds((chunk0 + c) * CHUNK, CHUNK), :],
                sem_out.at[b])

        for b in range(NBUF):
            idx_copy(b, b).start()

        def step2(i2, carry):
            for b in range(NBUF):
                c = i2 * NBUF + b
                idx_copy(c, b).wait()
                @pl.when(c >= NBUF)
                def _():
                    out_copy(c - NBUF, b).wait()
                gathers = [
                    pltpu.async_copy(
                        table_hbm.at[idx_v.at[b, j]],
                        rows_v.at[b, pl.ds(j * IDX_BLK, IDX_BLK), :],
                        sem_g)
                    for j in range(BLKS_PER_CHUNK)
                ]
                for g in gathers:
                    g.wait()
                out_copy(c, b).start()
                @pl.when(c + NBUF < n_chunks)
                def _():
                    idx_copy(c + NBUF, b).start()
            return carry

        lax.fori_loop(0, n_chunks // NBUF, step2, 0)
        for b in range(NBUF):
            out_copy(n_chunks - NBUF + b, b).wait()

    return k(table, idx2d)


def kernel(idx, W_train, W_frozen):
    table = jnp.concatenate([W_train, W_frozen], axis=0)
    idx2d = idx.reshape(-1, IDX_BLK)
    out = _sc_gather(table, idx2d)
    return out.reshape(idx.shape + (D,))
